# baseline (speedup 1.0000x reference)
import functools

import jax
import jax.numpy as jnp
from jax import lax
from jax.experimental import pallas as pl
from jax.experimental.pallas import tpu as pltpu
from jax.experimental.pallas import tpu_sc as plsc

VOCAB = 100000
SIZE = 64
PSIZE = 2 * SIZE
BATCH = 16384
W2 = 10
NEG = 5
NSENSE = NEG + 1
NCOL = 22

NC = 2
NS = 16
NW = NC * NS
BPW = BATCH // NW
CHUNK = 16
NCHUNK = BPW // CHUNK
LANES = 16
QV = SIZE // LANES

CTX_COLS = tuple(range(W2))
SENSE_COLS = (11, 12, 13, 14, 15, 16)
NROWS = 256


def _row_of_ctx(w, b):
    return w * CHUNK + b if w < 8 else 128 + (w - 8) * CHUNK + b


def _row_of_sense(j, b):
    return 160 + j * CHUNK + b


def _sc_body(dataT_hbm, gtab_hbm, stab_hbm, cw_hbm, out_hbm,
             stage, idxbuf, buf, cwbuf, outbuf, sem0, sem1):
    wid = lax.axis_index("s") * NC + lax.axis_index("c")
    base = wid * BPW
    sems = (sem0, sem1)

    pltpu.sync_copy(dataT_hbm.at[:, wid], stage)
    pltpu.sync_copy(cw_hbm, cwbuf)

    def transpose_body(c, _):
        sl = pl.ds(c * CHUNK, CHUNK)
        for k, col in enumerate(CTX_COLS[:8]):
            idxbuf[c, 0, pl.ds(k * CHUNK, CHUNK)] = stage[col, sl]
        for k, col in enumerate(CTX_COLS[8:]):
            idxbuf[c, 1, pl.ds(k * CHUNK, CHUNK)] = stage[col, sl]
        for k, col in enumerate(SENSE_COLS):
            idxbuf[c, 1, pl.ds(32 + k * CHUNK, CHUNK)] = stage[col, sl]
        return ()

    lax.fori_loop(0, NCHUNK, transpose_body, (), unroll=False)

    cwv = [[cwbuf[w, pl.ds(q * LANES, LANES)] for q in range(QV)]
           for w in range(W2)]
    lane = lax.broadcasted_iota(jnp.int32, (LANES,), 0)

    def copies(c, pa):
        return [
            pltpu.make_async_copy(gtab_hbm.at[idxbuf.at[c, 0]],
                                  buf.at[pa, pl.ds(0, 128)], sems[pa]),
            pltpu.make_async_copy(stab_hbm.at[idxbuf.at[c, 1]],
                                  buf.at[pa, pl.ds(128, 128)], sems[pa]),
        ]

    def fire(c, pa):
        for cp in copies(c, pa):
            cp.start()

    def drain(c, pa):
        for cp in copies(c, pa):
            cp.wait()

    def compute(c, pa):
        def body(b, ipvecs):
            sel = lane == b
            acc = []
            for q in range(QV):
                a = (buf[pa, _row_of_ctx(0, b), pl.ds(q * LANES, LANES)]
                     * cwv[0][q])
                for w in range(1, W2):
                    a = a + (buf[pa, _row_of_ctx(w, b),
                                 pl.ds(q * LANES, LANES)] * cwv[w][q])
                acc.append(a)
            new = []
            for j in range(NSENSE):
                row = _row_of_sense(j, b)
                p = buf[pa, row, pl.ds(0, LANES)] * acc[0]
                for q in range(1, QV):
                    p = p + buf[pa, row, pl.ds(q * LANES, LANES)] * acc[q]
                ip = plsc.cumsum(p)[LANES - 1]
                new.append(jnp.where(sel, ip, ipvecs[j]))
            return tuple(new)

        ips = lax.fori_loop(0, CHUNK, body,
                            tuple(jnp.zeros((LANES,), jnp.float32)
                                  for _ in range(NSENSE)),
                            unroll=False)
        for j in range(NSENSE):
            outbuf[j, :] = ips[j]
        pltpu.sync_copy(outbuf, out_hbm.at[:, pl.ds(base + c * CHUNK, CHUNK)])

    fire(0, 0)

    def pair_body(p, _):
        c0 = p * 2
        c1 = c0 + 1
        fire(c1, 1)
        drain(c0, 0)
        compute(c0, 0)
        drain(c1, 1)

        @pl.when(c1 + 1 < NCHUNK)
        def _():
            fire(c1 + 1, 0)

        compute(c1, 1)
        return ()

    lax.fori_loop(0, NCHUNK // 2, pair_body, (), unroll=False)


_sc_ips = functools.partial(
    pl.kernel,
    out_type=jax.ShapeDtypeStruct((NSENSE, BATCH), jnp.float32),
    mesh=plsc.VectorSubcoreMesh(core_axis_name="c", subcore_axis_name="s"),
    compiler_params=pltpu.CompilerParams(
        needs_layout_passes=False, use_tc_tiling_on_sc=False),
    scratch_types=[
        pltpu.VMEM((NCOL, BPW), jnp.int32),
        pltpu.VMEM((NCHUNK, 2, 128), jnp.int32),
        pltpu.VMEM((2, NROWS, PSIZE), jnp.float32),
        pltpu.VMEM((W2, SIZE), jnp.float32),
        pltpu.VMEM((NSENSE, CHUNK), jnp.float32),
        pltpu.SemaphoreType.DMA,
        pltpu.SemaphoreType.DMA,
    ],
)(_sc_body)


def _tc_loss_body(y_ref, m_ref, o_ref):
    y = y_ref[...]
    m = m_ref[...]
    pos = jnp.clip(y[0:1, :], -10.0, 10.0)
    neg = jnp.clip(y[1:NSENSE, :], -10.0, 10.0)
    pos_loss = jnp.sum(jnp.log1p(jnp.exp(-pos)), keepdims=True)
    neg_loss = jnp.sum(m * jnp.log1p(jnp.exp(neg)), keepdims=True)
    o_ref[...] = pos_loss + neg_loss


def kernel(data, global_embs, sense_embs, ctx_weight):
    gtab = jnp.pad(global_embs, ((0, 0), (0, PSIZE - SIZE)))
    stab = jnp.pad(sense_embs, ((0, 0), (0, PSIZE - SIZE)))
    dataT = data.T.reshape(NCOL, NW, BPW)
    maskf = data[:, W2 + 2 + NEG:].astype(jnp.float32).T

    ips = _sc_ips(dataT, gtab, stab, ctx_weight)

    out = pl.pallas_call(
        _tc_loss_body,
        out_shape=jax.ShapeDtypeStruct((1, 1), jnp.float32),
    )(ips, maskf)
    return out[0, 0]


# speedup vs baseline: 1.0713x; 1.0713x over previous
import functools

import jax
import jax.numpy as jnp
from jax import lax
from jax.experimental import pallas as pl
from jax.experimental.pallas import tpu as pltpu
from jax.experimental.pallas import tpu_sc as plsc

VOCAB = 100000
SIZE = 64
BATCH = 16384
W2 = 10
NEG = 5
NSENSE = NEG + 1
NCOL = 22

NC = 2
NS = 16
NW = NC * NS
BPW = BATCH // NW
CHUNK = 32
NCHUNK = BPW // CHUNK
LANES = 16
QV = SIZE // LANES

CTX_COLS = tuple(range(W2))
SENSE_COLS = (11, 12, 13, 14, 15, 16)


def _sc_body(dataT_hbm, gtab_hbm, stab_hbm, cw_hbm, out_hbm,
             idxbuf, gbuf, sbuf, cwbuf, outbuf, sem0, sem1):
    wid = lax.axis_index("s") * NC + lax.axis_index("c")
    base = wid * BPW
    sems = (sem0, sem1)

    pltpu.sync_copy(dataT_hbm.at[:, wid], idxbuf)
    pltpu.sync_copy(cw_hbm, cwbuf)

    cwv = [[cwbuf[w, pl.ds(q * LANES, LANES)] for q in range(QV)]
           for w in range(W2)]
    lane = lax.broadcasted_iota(jnp.int32, (LANES,), 0)

    def fire(c):
        pa = c & 1
        copies = []
        for k, col in enumerate(CTX_COLS):
            copies.append(pltpu.async_copy(
                gtab_hbm.at[idxbuf.at[col, c]], gbuf.at[pa, k], sems[pa]))
        for k, col in enumerate(SENSE_COLS):
            copies.append(pltpu.async_copy(
                stab_hbm.at[idxbuf.at[col, c]], sbuf.at[pa, k], sems[pa]))
        return copies

    inflight = fire(0)
    for c in range(NCHUNK):
        pa = c & 1
        for cp in inflight:
            cp.wait()
        if c + 1 < NCHUNK:
            inflight = fire(c + 1)

        def body(b, ipvecs):
            bi = b & (LANES - 1)
            sel = lane == bi
            acc = []
            for q in range(QV):
                a = gbuf[pa, 0, b, pl.ds(q * LANES, LANES)] * cwv[0][q]
                for w in range(1, W2):
                    a = a + gbuf[pa, w, b, pl.ds(q * LANES, LANES)] * cwv[w][q]
                acc.append(a)
            new = []
            for j in range(NSENSE):
                p = sbuf[pa, j, b, pl.ds(0, LANES)] * acc[0]
                for q in range(1, QV):
                    p = p + sbuf[pa, j, b, pl.ds(q * LANES, LANES)] * acc[q]
                ip = plsc.cumsum(p)[LANES - 1]
                new.append(jnp.where(sel, ip, ipvecs[j]))

            @pl.when(bi == LANES - 1)
            def _store():
                g0 = pl.multiple_of(b - (LANES - 1), LANES)
                for j in range(NSENSE):
                    outbuf[j, pl.ds(g0, LANES)] = new[j]

            return tuple(new)

        lax.fori_loop(0, CHUNK, body,
                      tuple(jnp.zeros((LANES,), jnp.float32)
                            for _ in range(NSENSE)),
                      unroll=False)
        pltpu.sync_copy(outbuf, out_hbm.at[:, pl.ds(base + c * CHUNK, CHUNK)])


_sc_ips = functools.partial(
    pl.kernel,
    out_type=jax.ShapeDtypeStruct((NSENSE, BATCH), jnp.float32),
    mesh=plsc.VectorSubcoreMesh(core_axis_name="c", subcore_axis_name="s"),
    compiler_params=pltpu.CompilerParams(
        needs_layout_passes=False, use_tc_tiling_on_sc=False),
    scratch_types=[
        pltpu.VMEM((NCOL, NCHUNK, CHUNK), jnp.int32),
        pltpu.VMEM((2, W2, CHUNK, SIZE), jnp.float32),
        pltpu.VMEM((2, NSENSE, CHUNK, SIZE), jnp.float32),
        pltpu.VMEM((W2, SIZE), jnp.float32),
        pltpu.VMEM((NSENSE, CHUNK), jnp.float32),
        pltpu.SemaphoreType.DMA,
        pltpu.SemaphoreType.DMA,
    ],
)(_sc_body)


def _tc_loss_body(y_ref, m_ref, o_ref):
    y = y_ref[...]
    m = m_ref[...]
    pos = jnp.clip(y[0:1, :], -10.0, 10.0)
    neg = jnp.clip(y[1:NSENSE, :], -10.0, 10.0)
    pos_loss = jnp.sum(jnp.log1p(jnp.exp(-pos)), keepdims=True)
    neg_loss = jnp.sum(m * jnp.log1p(jnp.exp(neg)), keepdims=True)
    o_ref[...] = pos_loss + neg_loss


def kernel(data, global_embs, sense_embs, ctx_weight):
    dataT = data.T.reshape(NCOL, NW, NCHUNK, CHUNK)
    maskf = data[:, W2 + 2 + NEG:].astype(jnp.float32).T

    ips = _sc_ips(dataT, global_embs, sense_embs, ctx_weight)

    out = pl.pallas_call(
        _tc_loss_body,
        out_shape=jax.ShapeDtypeStruct((1, 1), jnp.float32),
    )(ips, maskf)
    return out[0, 0]


# speedup vs baseline: 1.0728x; 1.0014x over previous
import functools

import jax
import jax.numpy as jnp
from jax import lax
from jax.experimental import pallas as pl
from jax.experimental.pallas import tpu as pltpu
from jax.experimental.pallas import tpu_sc as plsc

VOCAB = 100000
SIZE = 64
BATCH = 16384
W2 = 10
NEG = 5
NSENSE = NEG + 1
NCOL = 22

NC = 2
NS = 16
NW = NC * NS
BPW = BATCH // NW
CHUNK = 32
NCHUNK = BPW // CHUNK
LANES = 16
QV = SIZE // LANES

CTX_COLS = tuple(range(W2))
SENSE_COLS = (11, 12, 13, 14, 15, 16)


def _sc_body(dataT_hbm, gtab_hbm, stab_hbm, cw_hbm, out_hbm,
             idxbuf, gbuf, sbuf, cwbuf, outbuf, sem0, sem1):
    wid = lax.axis_index("s") * NC + lax.axis_index("c")
    base = wid * BPW
    sems = (sem0, sem1)

    pltpu.sync_copy(dataT_hbm.at[:, wid], idxbuf)
    pltpu.sync_copy(cw_hbm, cwbuf)

    cwv = [[cwbuf[w, pl.ds(q * LANES, LANES)] for q in range(QV)]
           for w in range(W2)]
    lane = lax.broadcasted_iota(jnp.int32, (LANES,), 0)

    def fire(c):
        pa = c & 1
        copies = []
        for k, col in enumerate(CTX_COLS):
            copies.append(pltpu.async_copy(
                gtab_hbm.at[idxbuf.at[col, c]], gbuf.at[pa, k], sems[pa]))
        for k, col in enumerate(SENSE_COLS):
            copies.append(pltpu.async_copy(
                stab_hbm.at[idxbuf.at[col, c]], sbuf.at[pa, k], sems[pa]))
        return copies

    inflight = fire(0)
    for c in range(NCHUNK):
        pa = c & 1
        for cp in inflight:
            cp.wait()
        if c + 1 < NCHUNK:
            inflight = fire(c + 1)

        def body(b, ipvecs):
            bi = b & (LANES - 1)
            sel = lane == bi
            acc = []
            for q in range(QV):
                a = gbuf[pa, 0, b, pl.ds(q * LANES, LANES)] * cwv[0][q]
                for w in range(1, W2):
                    a = a + gbuf[pa, w, b, pl.ds(q * LANES, LANES)] * cwv[w][q]
                acc.append(a)
            new = []
            for j in range(NSENSE):
                p = sbuf[pa, j, b, pl.ds(0, LANES)] * acc[0]
                for q in range(1, QV):
                    p = p + sbuf[pa, j, b, pl.ds(q * LANES, LANES)] * acc[q]
                ip = plsc.cumsum(p)[LANES - 1]
                new.append(jnp.where(sel, ip, ipvecs[j]))

            @pl.when(bi == LANES - 1)
            def _store():
                g0 = pl.multiple_of(b - (LANES - 1), LANES)
                for j in range(NSENSE):
                    outbuf[j, pl.ds(g0, LANES)] = new[j]

            return tuple(new)

        lax.fori_loop(0, CHUNK, body,
                      tuple(jnp.zeros((LANES,), jnp.float32)
                            for _ in range(NSENSE)),
                      unroll=False)
        pltpu.sync_copy(outbuf, out_hbm.at[:, pl.ds(base + c * CHUNK, CHUNK)])


_sc_ips = functools.partial(
    pl.kernel,
    out_type=jax.ShapeDtypeStruct((NSENSE, BATCH), jnp.float32),
    mesh=plsc.VectorSubcoreMesh(core_axis_name="c", subcore_axis_name="s"),
    compiler_params=pltpu.CompilerParams(
        needs_layout_passes=False, use_tc_tiling_on_sc=False),
    scratch_types=[
        pltpu.VMEM((NCOL, NCHUNK, CHUNK), jnp.int32),
        pltpu.VMEM((2, W2, CHUNK, SIZE), jnp.float32),
        pltpu.VMEM((2, NSENSE, CHUNK, SIZE), jnp.float32),
        pltpu.VMEM((W2, SIZE), jnp.float32),
        pltpu.VMEM((NSENSE, CHUNK), jnp.float32),
        pltpu.SemaphoreType.DMA,
        pltpu.SemaphoreType.DMA,
    ],
)(_sc_body)


def _tc_loss_body(y_ref, m_ref, o_ref):
    y = y_ref[...]
    m = m_ref[...]
    pos = jnp.clip(y[0:1, :], -10.0, 10.0)
    neg = jnp.clip(y[1:NSENSE, :], -10.0, 10.0)
    pos_loss = jnp.sum(jnp.log1p(jnp.exp(-pos)), keepdims=True)
    neg_loss = jnp.sum(m * jnp.log1p(jnp.exp(neg)), keepdims=True)
    o_ref[...] = pos_loss + neg_loss


def kernel(data, global_embs, sense_embs, ctx_weight):
    dataT = data.T.reshape(NCOL, NW, NCHUNK, CHUNK)
    maskf = data[:, W2 + 2 + NEG:].astype(jnp.float32).T
    stab = lax.optimization_barrier(
        sense_embs.reshape(-1)).reshape(2 * VOCAB, SIZE)

    ips = _sc_ips(dataT, global_embs, stab, ctx_weight)

    out = pl.pallas_call(
        _tc_loss_body,
        out_shape=jax.ShapeDtypeStruct((1, 1), jnp.float32),
    )(ips, maskf)
    return out[0, 0]


# speedup vs baseline: 1.1289x; 1.0523x over previous
import functools

import jax
import jax.numpy as jnp
from jax import lax
from jax.experimental import pallas as pl
from jax.experimental.pallas import tpu as pltpu
from jax.experimental.pallas import tpu_sc as plsc

VOCAB = 100000
SIZE = 64
BATCH = 16384
W2 = 10
NEG = 5
NSENSE = NEG + 1
NCOL = 22

NC = 2
NS = 16
NW = NC * NS
BPW = BATCH // NW
CHUNK = 32
NCHUNK = BPW // CHUNK
LANES = 16
QV = SIZE // LANES

CTX_COLS = tuple(range(W2))
SENSE_COLS = (11, 12, 13, 14, 15, 16)


def _worker_id():
    return lax.axis_index("s") * NC + lax.axis_index("c")


def _sc_ctx_body(idxT_hbm, gtab_hbm, cw_hbm, out_hbm,
                 idxbuf, gbuf, cwbuf, outbuf, sem0, sem1):
    wid = _worker_id()
    base = wid * BPW
    sems = (sem0, sem1)

    pltpu.sync_copy(idxT_hbm.at[:, wid], idxbuf)
    pltpu.sync_copy(cw_hbm, cwbuf)
    cwv = [[cwbuf[w, pl.ds(q * LANES, LANES)] for q in range(QV)]
           for w in range(W2)]

    def fire(c):
        pa = c & 1
        return [pltpu.async_copy(gtab_hbm.at[idxbuf.at[w, c]],
                                 gbuf.at[pa, w], sems[pa])
                for w in range(W2)]

    inflight = fire(0)
    for c in range(NCHUNK):
        pa = c & 1
        for cp in inflight:
            cp.wait()
        if c + 1 < NCHUNK:
            inflight = fire(c + 1)

        def body(b, _):
            for q in range(QV):
                a = gbuf[pa, 0, b, pl.ds(q * LANES, LANES)] * cwv[0][q]
                for w in range(1, W2):
                    a = a + gbuf[pa, w, b, pl.ds(q * LANES, LANES)] * cwv[w][q]
                outbuf[b, pl.ds(q * LANES, LANES)] = a
            return ()

        lax.fori_loop(0, CHUNK, body, (), unroll=False)
        pltpu.sync_copy(outbuf, out_hbm.at[pl.ds(base + c * CHUNK, CHUNK)])


_sc_ctx = functools.partial(
    pl.kernel,
    out_type=jax.ShapeDtypeStruct((BATCH, SIZE), jnp.float32),
    mesh=plsc.VectorSubcoreMesh(core_axis_name="c", subcore_axis_name="s"),
    compiler_params=pltpu.CompilerParams(
        needs_layout_passes=False, use_tc_tiling_on_sc=False),
    scratch_types=[
        pltpu.VMEM((W2, NCHUNK, CHUNK), jnp.int32),
        pltpu.VMEM((2, W2, CHUNK, SIZE), jnp.float32),
        pltpu.VMEM((W2, SIZE), jnp.float32),
        pltpu.VMEM((CHUNK, SIZE), jnp.float32),
        pltpu.SemaphoreType.DMA,
        pltpu.SemaphoreType.DMA,
    ],
)(_sc_ctx_body)


def _sc_dot_body(idxT_hbm, stab_hbm, ctxf_hbm, out_hbm,
                 idxbuf, sbuf, cbuf, outbuf, sem0, sem1):
    wid = _worker_id()
    base = wid * BPW
    sems = (sem0, sem1)

    pltpu.sync_copy(idxT_hbm.at[:, wid], idxbuf)
    lane = lax.broadcasted_iota(jnp.int32, (LANES,), 0)

    def fire(c):
        pa = c & 1
        copies = [pltpu.async_copy(stab_hbm.at[idxbuf.at[j, c]],
                                   sbuf.at[pa, j], sems[pa])
                  for j in range(NSENSE)]
        copies.append(pltpu.async_copy(
            ctxf_hbm.at[pl.ds(base + c * CHUNK, CHUNK)], cbuf.at[pa],
            sems[pa]))
        return copies

    inflight = fire(0)
    for c in range(NCHUNK):
        pa = c & 1
        for cp in inflight:
            cp.wait()
        if c + 1 < NCHUNK:
            inflight = fire(c + 1)

        def body(b, ipvecs):
            bi = b & (LANES - 1)
            sel = lane == bi
            acc = [cbuf[pa, b, pl.ds(q * LANES, LANES)] for q in range(QV)]
            new = []
            for j in range(NSENSE):
                p = sbuf[pa, j, b, pl.ds(0, LANES)] * acc[0]
                for q in range(1, QV):
                    p = p + sbuf[pa, j, b, pl.ds(q * LANES, LANES)] * acc[q]
                ip = plsc.cumsum(p)[LANES - 1]
                new.append(jnp.where(sel, ip, ipvecs[j]))

            @pl.when(bi == LANES - 1)
            def _store():
                g0 = pl.multiple_of(b - (LANES - 1), LANES)
                for j in range(NSENSE):
                    outbuf[j, pl.ds(g0, LANES)] = new[j]

            return tuple(new)

        lax.fori_loop(0, CHUNK, body,
                      tuple(jnp.zeros((LANES,), jnp.float32)
                            for _ in range(NSENSE)),
                      unroll=False)
        pltpu.sync_copy(outbuf, out_hbm.at[:, pl.ds(base + c * CHUNK, CHUNK)])


_sc_dot = functools.partial(
    pl.kernel,
    out_type=jax.ShapeDtypeStruct((NSENSE, BATCH), jnp.float32),
    mesh=plsc.VectorSubcoreMesh(core_axis_name="c", subcore_axis_name="s"),
    compiler_params=pltpu.CompilerParams(
        needs_layout_passes=False, use_tc_tiling_on_sc=False),
    scratch_types=[
        pltpu.VMEM((NSENSE, NCHUNK, CHUNK), jnp.int32),
        pltpu.VMEM((2, NSENSE, CHUNK, SIZE), jnp.float32),
        pltpu.VMEM((2, CHUNK, SIZE), jnp.float32),
        pltpu.VMEM((NSENSE, CHUNK), jnp.float32),
        pltpu.SemaphoreType.DMA,
        pltpu.SemaphoreType.DMA,
    ],
)(_sc_dot_body)


def _tc_loss_body(y_ref, m_ref, o_ref):
    y = y_ref[...]
    m = m_ref[...]
    pos = jnp.clip(y[0:1, :], -10.0, 10.0)
    neg = jnp.clip(y[1:NSENSE, :], -10.0, 10.0)
    pos_loss = jnp.sum(jnp.log1p(jnp.exp(-pos)), keepdims=True)
    neg_loss = jnp.sum(m * jnp.log1p(jnp.exp(neg)), keepdims=True)
    o_ref[...] = pos_loss + neg_loss


def kernel(data, global_embs, sense_embs, ctx_weight):
    dataT = data.T
    ctxT = dataT[:W2].reshape(W2, NW, NCHUNK, CHUNK)
    senseT = dataT[W2 + 1:W2 + 1 + NSENSE].reshape(NSENSE, NW, NCHUNK, CHUNK)
    maskf = data[:, W2 + 2 + NEG:].astype(jnp.float32).T

    ctxf = _sc_ctx(ctxT, global_embs, ctx_weight)
    ips = _sc_dot(senseT, sense_embs, ctxf)

    out = pl.pallas_call(
        _tc_loss_body,
        out_shape=jax.ShapeDtypeStruct((1, 1), jnp.float32),
    )(ips, maskf)
    return out[0, 0]


# speedup vs baseline: 1.1410x; 1.0107x over previous
import functools

import jax
import jax.numpy as jnp
from jax import lax
from jax.experimental import pallas as pl
from jax.experimental.pallas import tpu as pltpu
from jax.experimental.pallas import tpu_sc as plsc

VOCAB = 100000
SIZE = 64
BATCH = 16384
W2 = 10
NEG = 5
NSENSE = NEG + 1
NCOL = 22

NC = 2
NS = 16
NW = NC * NS
BPW = BATCH // NW
CHUNK = 32
NCHUNK = BPW // CHUNK
CHUNK2 = 64
NCHUNK2 = BPW // CHUNK2
LANES = 16
QV = SIZE // LANES

CTX_COLS = tuple(range(W2))
SENSE_COLS = (11, 12, 13, 14, 15, 16)


def _worker_id():
    return lax.axis_index("s") * NC + lax.axis_index("c")


def _sc_ctx_body(idxT_hbm, gtab_hbm, cw_hbm, out_hbm,
                 idxbuf, gbuf, cwbuf, outbuf, sem0, sem1):
    wid = _worker_id()
    base = wid * BPW
    sems = (sem0, sem1)

    pltpu.sync_copy(idxT_hbm.at[:, wid], idxbuf)
    pltpu.sync_copy(cw_hbm, cwbuf)
    cwv = [[cwbuf[w, pl.ds(q * LANES, LANES)] for q in range(QV)]
           for w in range(W2)]

    def fire(c):
        pa = c & 1
        return [pltpu.async_copy(gtab_hbm.at[idxbuf.at[w, c]],
                                 gbuf.at[pa, w], sems[pa])
                for w in range(W2)]

    inflight = fire(0)
    for c in range(NCHUNK):
        pa = c & 1
        for cp in inflight:
            cp.wait()
        if c + 1 < NCHUNK:
            inflight = fire(c + 1)

        def body(b, _):
            for q in range(QV):
                a = gbuf[pa, 0, b, pl.ds(q * LANES, LANES)] * cwv[0][q]
                for w in range(1, W2):
                    a = a + gbuf[pa, w, b, pl.ds(q * LANES, LANES)] * cwv[w][q]
                outbuf[b, pl.ds(q * LANES, LANES)] = a
            return ()

        lax.fori_loop(0, CHUNK, body, (), unroll=False)
        pltpu.sync_copy(outbuf, out_hbm.at[pl.ds(base + c * CHUNK, CHUNK)])


_sc_ctx = functools.partial(
    pl.kernel,
    out_type=jax.ShapeDtypeStruct((BATCH, SIZE), jnp.float32),
    mesh=plsc.VectorSubcoreMesh(core_axis_name="c", subcore_axis_name="s"),
    compiler_params=pltpu.CompilerParams(
        needs_layout_passes=False, use_tc_tiling_on_sc=False),
    scratch_types=[
        pltpu.VMEM((W2, NCHUNK, CHUNK), jnp.int32),
        pltpu.VMEM((2, W2, CHUNK, SIZE), jnp.float32),
        pltpu.VMEM((W2, SIZE), jnp.float32),
        pltpu.VMEM((CHUNK, SIZE), jnp.float32),
        pltpu.SemaphoreType.DMA,
        pltpu.SemaphoreType.DMA,
    ],
)(_sc_ctx_body)


def _sc_dot_body(idxT_hbm, stab_hbm, ctxf_hbm, out_hbm,
                 idxbuf, sbuf, cbuf, outbuf, sem0, sem1):
    wid = _worker_id()
    base = wid * BPW
    sems = (sem0, sem1)

    pltpu.sync_copy(idxT_hbm.at[:, wid], idxbuf)
    lane = lax.broadcasted_iota(jnp.int32, (LANES,), 0)

    def fire(c):
        pa = c & 1
        copies = [pltpu.async_copy(stab_hbm.at[idxbuf.at[j, c]],
                                   sbuf.at[pa, j], sems[pa])
                  for j in range(NSENSE)]
        copies.append(pltpu.async_copy(
            ctxf_hbm.at[pl.ds(base + c * CHUNK2, CHUNK2)], cbuf.at[pa],
            sems[pa]))
        return copies

    inflight = fire(0)
    for c in range(NCHUNK2):
        pa = c & 1
        for cp in inflight:
            cp.wait()
        if c + 1 < NCHUNK2:
            inflight = fire(c + 1)

        def body(b, ipvecs):
            bi = b & (LANES - 1)
            sel = lane == bi
            acc = [cbuf[pa, b, pl.ds(q * LANES, LANES)] for q in range(QV)]
            new = []
            for j in range(NSENSE):
                p = sbuf[pa, j, b, pl.ds(0, LANES)] * acc[0]
                for q in range(1, QV):
                    p = p + sbuf[pa, j, b, pl.ds(q * LANES, LANES)] * acc[q]
                ip = plsc.cumsum(p)[LANES - 1]
                new.append(jnp.where(sel, ip, ipvecs[j]))

            @pl.when(bi == LANES - 1)
            def _store():
                g0 = pl.multiple_of(b - (LANES - 1), LANES)
                for j in range(NSENSE):
                    outbuf[j, pl.ds(g0, LANES)] = new[j]

            return tuple(new)

        lax.fori_loop(0, CHUNK2, body,
                      tuple(jnp.zeros((LANES,), jnp.float32)
                            for _ in range(NSENSE)),
                      unroll=False)
        pltpu.sync_copy(outbuf, out_hbm.at[:, pl.ds(base + c * CHUNK2, CHUNK2)])


_sc_dot = functools.partial(
    pl.kernel,
    out_type=jax.ShapeDtypeStruct((NSENSE, BATCH), jnp.float32),
    mesh=plsc.VectorSubcoreMesh(core_axis_name="c", subcore_axis_name="s"),
    compiler_params=pltpu.CompilerParams(
        needs_layout_passes=False, use_tc_tiling_on_sc=False),
    scratch_types=[
        pltpu.VMEM((NSENSE, NCHUNK2, CHUNK2), jnp.int32),
        pltpu.VMEM((2, NSENSE, CHUNK2, SIZE), jnp.float32),
        pltpu.VMEM((2, CHUNK2, SIZE), jnp.float32),
        pltpu.VMEM((NSENSE, CHUNK2), jnp.float32),
        pltpu.SemaphoreType.DMA,
        pltpu.SemaphoreType.DMA,
    ],
)(_sc_dot_body)


def _tc_loss_body(y_ref, m_ref, o_ref):
    y = y_ref[...]
    m = m_ref[...]
    pos = jnp.clip(y[0:1, :], -10.0, 10.0)
    neg = jnp.clip(y[1:NSENSE, :], -10.0, 10.0)
    pos_loss = jnp.sum(jnp.log1p(jnp.exp(-pos)), keepdims=True)
    neg_loss = jnp.sum(m * jnp.log1p(jnp.exp(neg)), keepdims=True)
    o_ref[...] = pos_loss + neg_loss


def kernel(data, global_embs, sense_embs, ctx_weight):
    dataT = data.T
    ctxT = dataT[:W2].reshape(W2, NW, NCHUNK, CHUNK)
    senseT = dataT[W2 + 1:W2 + 1 + NSENSE].reshape(NSENSE, NW, NCHUNK2, CHUNK2)
    maskf = data[:, W2 + 2 + NEG:].astype(jnp.float32).T

    ctxf = _sc_ctx(ctxT, global_embs, ctx_weight)
    ips = _sc_dot(senseT, sense_embs, ctxf)

    out = pl.pallas_call(
        _tc_loss_body,
        out_shape=jax.ShapeDtypeStruct((1, 1), jnp.float32),
    )(ips, maskf)
    return out[0, 0]
